# Initial kernel scaffold; baseline (speedup 1.0000x reference)
import functools

import jax
import jax.numpy as jnp
from jax import lax
from jax.experimental import pallas as pl
from jax.experimental.pallas import tpu as pltpu
from jax.experimental.pallas import tpu_sc as plsc


_B = 16384
_F = 26
_V = 100000
_D = 16

_NC = 2
_NS = 16
_NW = _NC * _NS

_N = _B * _F
_IPV = 128
_NVEC = _N // _IPV
_VEC_W = _NVEC // _NW
_ROWS_W = _N // _NW

_GPC = 8
_CHUNK = _GPC * _IPV
_NSTEP = _ROWS_W // _CHUNK


def _build_gather():
    mesh = plsc.VectorSubcoreMesh(core_axis_name="c", subcore_axis_name="s")

    @functools.partial(
        pl.kernel,
        mesh=mesh,
        out_type=jax.ShapeDtypeStruct((_N, _D), jnp.float32),
        scratch_types=[
            pltpu.VMEM((_VEC_W, _IPV), jnp.int32),
            pltpu.VMEM((_CHUNK, _D), jnp.float32),
            pltpu.SemaphoreType.DMA,
        ],
    )
    def gather(table_hbm, idx_hbm, out_hbm, idx_v, rows_v, sem):
        wid = lax.axis_index("s") * _NC + lax.axis_index("c")
        pltpu.sync_copy(idx_hbm.at[pl.ds(wid * _VEC_W, _VEC_W)], idx_v)

        def step(g, carry):
            copies = [
                pltpu.async_copy(
                    table_hbm.at[idx_v.at[g * _GPC + j]],
                    rows_v.at[pl.ds(j * _IPV, _IPV)],
                    sem,
                )
                for j in range(_GPC)
            ]
            for c in copies:
                c.wait()
            pltpu.sync_copy(
                rows_v,
                out_hbm.at[pl.ds(wid * _ROWS_W + g * _CHUNK, _CHUNK)],
            )
            return carry

        lax.fori_loop(0, _NSTEP, step, 0)

    return gather


_gather = _build_gather()


def kernel(x, tables):
    batch, num_fields = x.shape
    flat_idx = (
        x.astype(jnp.int32)
        + (jnp.arange(num_fields, dtype=jnp.int32) * _V)[None, :]
    ).reshape(_NVEC, _IPV)
    table2d = tables.reshape(num_fields * _V, _D)
    rows = _gather(table2d, flat_idx)
    return rows.reshape(batch, num_fields * _D)


# speedup vs baseline: 1.2561x; 1.2561x over previous
import functools

import jax
import jax.numpy as jnp
from jax import lax
from jax.experimental import pallas as pl
from jax.experimental.pallas import tpu as pltpu
from jax.experimental.pallas import tpu_sc as plsc


_B = 16384
_F = 26
_V = 100000
_D = 16

_NC = 2
_NS = 16
_NW = _NC * _NS

_N = _B * _F
_IPV = 128
_NVEC = _N // _IPV
_VEC_W = _NVEC // _NW
_ROWS_W = _N // _NW

_GPC = 8
_CHUNK = _GPC * _IPV
_NSTEP = _ROWS_W // _CHUNK


def _build_gather():
    mesh = plsc.VectorSubcoreMesh(core_axis_name="c", subcore_axis_name="s")

    @functools.partial(
        pl.kernel,
        mesh=mesh,
        out_type=jax.ShapeDtypeStruct((_N, _D), jnp.float32),
        scratch_types=[
            pltpu.VMEM((_VEC_W, _IPV), jnp.int32),
            pltpu.VMEM((_CHUNK, _D), jnp.float32),
            pltpu.SemaphoreType.DMA,
        ],
        compiler_params=pltpu.CompilerParams(use_tc_tiling_on_sc=False),
    )
    def gather(table_hbm, idx_hbm, out_hbm, idx_v, rows_v, sem):
        wid = lax.axis_index("s") * _NC + lax.axis_index("c")
        pltpu.sync_copy(idx_hbm.at[pl.ds(wid * _VEC_W, _VEC_W)], idx_v)

        def step(g, carry):
            copies = [
                pltpu.async_copy(
                    table_hbm.at[idx_v.at[g * _GPC + j]],
                    rows_v.at[pl.ds(j * _IPV, _IPV)],
                    sem,
                )
                for j in range(_GPC)
            ]
            for c in copies:
                c.wait()
            pltpu.sync_copy(
                rows_v,
                out_hbm.at[pl.ds(wid * _ROWS_W + g * _CHUNK, _CHUNK)],
            )
            return carry

        lax.fori_loop(0, _NSTEP, step, 0)

    return gather


_gather = _build_gather()


def kernel(x, tables):
    batch, num_fields = x.shape
    flat_idx = (
        x.astype(jnp.int32)
        + (jnp.arange(num_fields, dtype=jnp.int32) * _V)[None, :]
    ).reshape(_NVEC, _IPV)
    table2d = tables.reshape(num_fields * _V, _D)
    rows = _gather(table2d, flat_idx)
    return rows.reshape(batch, num_fields * _D)


# speedup vs baseline: 5.6585x; 4.5048x over previous
import functools

import jax
import jax.numpy as jnp
from jax import lax
from jax.experimental import pallas as pl
from jax.experimental.pallas import tpu as pltpu
from jax.experimental.pallas import tpu_sc as plsc


_B = 16384
_F = 26
_V = 100000
_D = 16

_NC = 2
_NS = 16
_NW = _NC * _NS

_NFD = _F * _D
_TPW = _NFD // _NW
_QB = _B // 4
_LOOKS = _QB // 16


def _build_gather():
    mesh = plsc.VectorSubcoreMesh(core_axis_name="c", subcore_axis_name="s")

    @functools.partial(
        pl.kernel,
        mesh=mesh,
        out_type=jax.ShapeDtypeStruct((_NFD, _B), jnp.float32),
        scratch_types=[
            pltpu.VMEM((_V,), jnp.float32),
            pltpu.VMEM((_QB,), jnp.int32),
            pltpu.VMEM((_QB,), jnp.float32),
            pltpu.SemaphoreType.DMA,
        ],
        compiler_params=pltpu.CompilerParams(
            use_tc_tiling_on_sc=True, needs_layout_passes=False
        ),
    )
    def gather_t(tab_hbm, xt_hbm, out_hbm, voc_v, idx_v, row_v, sem):
        wid = lax.axis_index("s") * _NC + lax.axis_index("c")

        def task(t, carry):
            fd = wid * _TPW + t
            f = fd // _D
            d = fd % _D
            pltpu.sync_copy(tab_hbm.at[f, d], voc_v)

            def quarter(q, c1):
                pltpu.sync_copy(xt_hbm.at[f, pl.ds(q * _QB, _QB)], idx_v)

                @pl.loop(0, _LOOKS, unroll=16)
                def look(i):
                    iv = idx_v[pl.ds(i * 16, 16)]
                    row_v[pl.ds(i * 16, 16)] = plsc.load_gather(voc_v, [iv])
                pltpu.sync_copy(row_v, out_hbm.at[fd, pl.ds(q * _QB, _QB)])
                return c1

            lax.fori_loop(0, 4, quarter, 0)
            return carry

        lax.fori_loop(0, _TPW, task, 0)

    return gather_t


_gather_t = _build_gather()


def kernel(x, tables):
    batch, num_fields = x.shape
    tab_t = jnp.transpose(tables, (0, 2, 1))
    x_t = jnp.transpose(x, (1, 0)).astype(jnp.int32)
    out = _gather_t(tab_t, x_t)
    return jnp.transpose(out, (1, 0)).reshape(batch, num_fields * _D)
